# Initial kernel scaffold; baseline (speedup 1.0000x reference)
import jax
import jax.numpy as jnp
from jax.experimental import pallas as pl


def kernel(pred, all_gt_grasps, gt_counts):
    raise NotImplementedError("write your pallas kernel here")


# speedup vs baseline: 82.4062x; 82.4062x over previous
import jax
import jax.numpy as jnp
from jax import lax
from jax.experimental import pallas as pl
from jax.experimental.pallas import tpu as pltpu
from jax.experimental.pallas import tpu_sc as plsc

N = 8192
M = 57344
KMAX = 7
NC, NS, L = 2, 16, 16
NW = NC * NS
PW = N // NW
ROWS = PW * KMAX

SIGMA = 0.15
NEG_INV_2SIG2 = -1.0 / (2.0 * SIGMA * SIGMA)


def _log1p_series(z):
    u = z / (2.0 + z)
    u2 = u * u
    p = 1.0 / 9.0
    p = p * u2 + 1.0 / 7.0
    p = p * u2 + 1.0 / 5.0
    p = p * u2 + 1.0 / 3.0
    p = p * u2 + 1.0
    return 2.0 * u * p


def _body(gt_hbm, predf_hbm, off_hbm, cnt_hbm, out_hbm,
          off_v, cnt_v, predf_v, rows_v, acc_v):
    wid = lax.axis_index("s") * NC + lax.axis_index("c")
    base = wid * PW

    pltpu.sync_copy(off_hbm.at[pl.ds(base, PW)], off_v)
    pltpu.sync_copy(cnt_hbm.at[pl.ds(base, PW)], cnt_v)
    for k in range(6):
        pltpu.sync_copy(predf_hbm.at[pl.ds(k * N + base, PW)],
                        predf_v.at[pl.ds(k * PW, PW)])

    ws = off_v[pl.ds(0, L)][0]
    ws8 = pl.multiple_of(ws * 8, 8)
    pltpu.sync_copy(gt_hbm.at[pl.ds(ws8, ROWS * 8)], rows_v)

    iota = lax.iota(jnp.int32, L)

    def step(t, carry):
        s_pos, s_ang, s_wid, s_sco = carry
        cnt16 = cnt_v[pl.ds(t * L, L)]
        o8 = off_v[pl.ds(t * L, L)] * 8 - ws8
        p0 = predf_v[pl.ds(0 * PW + t * L, L)]
        p1 = predf_v[pl.ds(1 * PW + t * L, L)]
        p2 = predf_v[pl.ds(2 * PW + t * L, L)]
        p3 = predf_v[pl.ds(3 * PW + t * L, L)]
        p4 = predf_v[pl.ds(4 * PW + t * L, L)]
        p5 = predf_v[pl.ds(5 * PW + t * L, L)]

        best_sq = jnp.full((L,), jnp.inf, jnp.float32)
        best_fx = jnp.zeros((L,), jnp.int32)
        for j in range(KMAX):
            valid = j < cnt16
            fx = o8 + (j * 8)
            gx = plsc.load_gather(rows_v, [fx])
            gy = plsc.load_gather(rows_v, [fx + 1])
            dx = p0 - gx
            dy = p1 - gy
            sq = dx * dx + dy * dy
            better = valid & (sq < best_sq)
            best_sq = jnp.where(better, sq, best_sq)
            best_fx = jnp.where(better, fx, best_fx)

        has = cnt16 > 0
        zero = jnp.zeros((L,), jnp.float32)
        lp = jnp.where(has, 1.0 - jnp.exp(best_sq * NEG_INV_2SIG2), zero)

        g2 = plsc.load_gather(rows_v, [best_fx + 2])
        g3 = plsc.load_gather(rows_v, [best_fx + 3])
        g4 = plsc.load_gather(rows_v, [best_fx + 4])
        g5 = plsc.load_gather(rows_v, [best_fx + 5])

        la = jnp.where(has, jnp.abs(p2 - g2) + jnp.abs(p3 - g3), zero)
        d = p4 - g4
        ad = jnp.abs(d)
        lw = jnp.where(has, jnp.where(ad < 1.0, 0.5 * d * d, ad - 0.5), zero)
        x = p5
        bce = jnp.maximum(x, zero) - x * g5 + _log1p_series(jnp.exp(-jnp.abs(x)))
        ls = jnp.where(has & (g5 > 0), bce, zero)

        return (s_pos + lp, s_ang + la, s_wid + lw, s_sco + ls)

    zeros = jnp.zeros((L,), jnp.float32)
    s_pos, s_ang, s_wid, s_sco = lax.fori_loop(
        0, PW // L, step, (zeros, zeros, zeros, zeros))

    acc_v[pl.ds(0, L)] = s_pos
    acc_v[pl.ds(L, L)] = s_ang
    acc_v[pl.ds(2 * L, L)] = s_wid
    acc_v[pl.ds(3 * L, L)] = s_sco
    pltpu.sync_copy(acc_v, out_hbm.at[pl.ds(wid * 4 * L, 4 * L)])


@jax.jit
def kernel(pred, all_gt_grasps, gt_counts):
    counts = gt_counts.astype(jnp.int32)
    offsets = jnp.cumsum(counts) - counts
    predf = pred.T.reshape(6 * N)
    gt_flat = jnp.pad(all_gt_grasps, ((0, 0), (0, 2))).reshape(M * 8)

    sc_kernel = pl.kernel(
        _body,
        out_type=jax.ShapeDtypeStruct((NW * 4 * L,), jnp.float32),
        mesh=plsc.VectorSubcoreMesh(core_axis_name="c", subcore_axis_name="s",
                                    num_cores=NC, num_subcores=NS),
        compiler_params=pltpu.CompilerParams(needs_layout_passes=False),
        scratch_types=[
            pltpu.VMEM((PW,), jnp.int32),
            pltpu.VMEM((PW,), jnp.int32),
            pltpu.VMEM((6 * PW,), jnp.float32),
            pltpu.VMEM((ROWS * 8,), jnp.float32),
            pltpu.VMEM((4 * L,), jnp.float32),
        ],
    )
    partials = sc_kernel(gt_flat, predf, offsets, counts)
    sums = partials.reshape(NW, 4, L).sum(axis=(0, 2))
    lp = sums[0] / N
    la = sums[1] / N
    lw = sums[2] / N
    ls = sums[3] / N
    lg = lp + la + lw + 0.5 * ls
    return jnp.stack([lp, la, lw, ls, lg])


# speedup vs baseline: 97.6516x; 1.1850x over previous
import jax
import jax.numpy as jnp
from jax import lax
from jax.experimental import pallas as pl
from jax.experimental.pallas import tpu as pltpu
from jax.experimental.pallas import tpu_sc as plsc

N = 8192
M = 57344
KMAX = 7
NC, NS, L = 2, 16, 16
NW = NC * NS
PW = N // NW
ROWS = PW * KMAX

SIGMA = 0.15
NEG_INV_2SIG2 = -1.0 / (2.0 * SIGMA * SIGMA)


def _log1p_series(z):
    u = z / (2.0 + z)
    u2 = u * u
    p = 1.0 / 9.0
    p = p * u2 + 1.0 / 7.0
    p = p * u2 + 1.0 / 5.0
    p = p * u2 + 1.0 / 3.0
    p = p * u2 + 1.0
    return 2.0 * u * p


WLEN = ROWS * 6 + 8


def _body(gt_hbm, pred_hbm, off_hbm, cnt_hbm, out_hbm,
          off_v, cnt_v, predc_v, rows_v, acc_v):
    wid = lax.axis_index("s") * NC + lax.axis_index("c")
    base = wid * PW

    pltpu.sync_copy(off_hbm.at[pl.ds(base, PW)], off_v)
    pltpu.sync_copy(cnt_hbm.at[pl.ds(base, PW)], cnt_v)
    pltpu.sync_copy(pred_hbm.at[pl.ds(base * 6, PW * 6)], predc_v)

    ws = off_v[pl.ds(0, L)][0]
    a8 = jnp.minimum((ws * 6 // 8) * 8, M * 6 - WLEN)
    a8 = pl.multiple_of(a8, 8)
    pltpu.sync_copy(gt_hbm.at[pl.ds(a8, WLEN)], rows_v)

    iota = lax.iota(jnp.int32, L)
    iota6 = iota * 6

    def step(t, carry):
        s_pos, s_ang, s_wid, s_sco = carry
        cnt16 = cnt_v[pl.ds(t * L, L)]
        o6 = off_v[pl.ds(t * L, L)] * 6 - a8
        pbase = iota6 + t * (L * 6)
        p0 = plsc.load_gather(predc_v, [pbase])
        p1 = plsc.load_gather(predc_v, [pbase + 1])
        p2 = plsc.load_gather(predc_v, [pbase + 2])
        p3 = plsc.load_gather(predc_v, [pbase + 3])
        p4 = plsc.load_gather(predc_v, [pbase + 4])
        p5 = plsc.load_gather(predc_v, [pbase + 5])

        best_sq = jnp.full((L,), jnp.inf, jnp.float32)
        best_fx = jnp.zeros((L,), jnp.int32)
        for j in range(KMAX):
            valid = j < cnt16
            fx = o6 + (j * 6)
            gx = plsc.load_gather(rows_v, [fx])
            gy = plsc.load_gather(rows_v, [fx + 1])
            dx = p0 - gx
            dy = p1 - gy
            sq = dx * dx + dy * dy
            better = valid & (sq < best_sq)
            best_sq = jnp.where(better, sq, best_sq)
            best_fx = jnp.where(better, fx, best_fx)

        has = cnt16 > 0
        zero = jnp.zeros((L,), jnp.float32)
        lp = jnp.where(has, 1.0 - jnp.exp(best_sq * NEG_INV_2SIG2), zero)

        g2 = plsc.load_gather(rows_v, [best_fx + 2])
        g3 = plsc.load_gather(rows_v, [best_fx + 3])
        g4 = plsc.load_gather(rows_v, [best_fx + 4])
        g5 = plsc.load_gather(rows_v, [best_fx + 5])

        la = jnp.where(has, jnp.abs(p2 - g2) + jnp.abs(p3 - g3), zero)
        d = p4 - g4
        ad = jnp.abs(d)
        lw = jnp.where(has, jnp.where(ad < 1.0, 0.5 * d * d, ad - 0.5), zero)
        x = p5
        bce = jnp.maximum(x, zero) - x * g5 + _log1p_series(jnp.exp(-jnp.abs(x)))
        ls = jnp.where(has & (g5 > 0), bce, zero)

        return (s_pos + lp, s_ang + la, s_wid + lw, s_sco + ls)

    zeros = jnp.zeros((L,), jnp.float32)
    s_pos, s_ang, s_wid, s_sco = lax.fori_loop(
        0, PW // L, step, (zeros, zeros, zeros, zeros))

    acc_v[pl.ds(0, L)] = s_pos
    acc_v[pl.ds(L, L)] = s_ang
    acc_v[pl.ds(2 * L, L)] = s_wid
    acc_v[pl.ds(3 * L, L)] = s_sco
    pltpu.sync_copy(acc_v, out_hbm.at[pl.ds(wid * 4 * L, 4 * L)])


@jax.jit
def kernel(pred, all_gt_grasps, gt_counts):
    counts = gt_counts.astype(jnp.int32)
    offsets = jnp.cumsum(counts) - counts
    pred_flat = pred.reshape(6 * N)
    gt_flat = all_gt_grasps.reshape(M * 6)

    sc_kernel = pl.kernel(
        _body,
        out_type=jax.ShapeDtypeStruct((NW * 4 * L,), jnp.float32),
        mesh=plsc.VectorSubcoreMesh(core_axis_name="c", subcore_axis_name="s",
                                    num_cores=NC, num_subcores=NS),
        compiler_params=pltpu.CompilerParams(needs_layout_passes=False),
        scratch_types=[
            pltpu.VMEM((PW,), jnp.int32),
            pltpu.VMEM((PW,), jnp.int32),
            pltpu.VMEM((6 * PW,), jnp.float32),
            pltpu.VMEM((WLEN,), jnp.float32),
            pltpu.VMEM((4 * L,), jnp.float32),
        ],
    )
    partials = sc_kernel(gt_flat, pred_flat, offsets, counts)
    sums = partials.reshape(NW, 4, L).sum(axis=(0, 2))
    lp = sums[0] / N
    la = sums[1] / N
    lw = sums[2] / N
    ls = sums[3] / N
    lg = lp + la + lw + 0.5 * ls
    return jnp.stack([lp, la, lw, ls, lg])


# speedup vs baseline: 212.1309x; 2.1723x over previous
import jax
import jax.numpy as jnp
from jax import lax
from jax.experimental import pallas as pl
from jax.experimental.pallas import tpu as pltpu
from jax.experimental.pallas import tpu_sc as plsc

N = 8192
M = 57344
KMAX = 7
NC, NS, L = 2, 16, 16
NW = NC * NS
PW = N // NW
ROWS = PW * KMAX
WIN = ROWS + 8

SIGMA = 0.15
NEG_INV_2SIG2 = -1.0 / (2.0 * SIGMA * SIGMA)


def _log1p_series(z):
    u = z / (2.0 + z)
    u2 = u * u
    p = 1.0 / 9.0
    p = p * u2 + 1.0 / 7.0
    p = p * u2 + 1.0 / 5.0
    p = p * u2 + 1.0 / 3.0
    p = p * u2 + 1.0
    return 2.0 * u * p


def _body(gt_hbm, predt_hbm, cnt_hbm, out_hbm,
          cnt_v, predt_v, rows_v, scan_v, acc_v, sem):
    wid = lax.axis_index("s") * NC + lax.axis_index("c")
    base = wid * PW

    d_cnt = pltpu.async_copy(cnt_hbm, cnt_v, sem)
    descs = []
    for k in range(6):
        descs.append(pltpu.async_copy(predt_hbm.at[pl.ds(k * N + base, PW)],
                                      predt_v.at[pl.ds(k * PW, PW)], sem))
    scan_v[pl.ds(0, L)] = jnp.zeros((L,), jnp.int32)
    d_cnt.wait()

    def acc_step(c, v):
        return v + cnt_v[pl.ds(c * L, L)]

    accv = lax.fori_loop(0, wid * (PW // L), acc_step,
                         jnp.zeros((L,), jnp.int32))
    ws = accv[0]
    for l in range(1, L):
        ws = ws + accv[l]

    a8 = jnp.minimum((ws // 8) * 8, M - WIN)
    a8 = pl.multiple_of(a8, 8)
    for k in range(6):
        descs.append(pltpu.async_copy(gt_hbm.at[pl.ds(k * M + a8, WIN)],
                                      rows_v.at[pl.ds(k * WIN, WIN)], sem))
    for d in descs:
        d.wait()

    def step(t, carry):
        s_pos, s_ang, s_wid, s_sco, carry_off = carry
        cnt16 = cnt_v[pl.ds(base + t * L, L)]

        s = cnt16
        for dsh in (1, 2, 4, 8):
            scan_v[pl.ds(L // 2, L)] = s
            s = s + scan_v[pl.ds(L // 2 - dsh, L)]
        off16 = (carry_off + s) - cnt16
        carry_off = carry_off + s[L - 1]

        o = off16 - a8
        p0 = predt_v[pl.ds(0 * PW + t * L, L)]
        p1 = predt_v[pl.ds(1 * PW + t * L, L)]
        p2 = predt_v[pl.ds(2 * PW + t * L, L)]
        p3 = predt_v[pl.ds(3 * PW + t * L, L)]
        p4 = predt_v[pl.ds(4 * PW + t * L, L)]
        p5 = predt_v[pl.ds(5 * PW + t * L, L)]

        best_sq = jnp.full((L,), jnp.inf, jnp.float32)
        best_o = jnp.zeros((L,), jnp.int32)
        for j in range(KMAX):
            valid = j < cnt16
            oj = o + j
            gx = plsc.load_gather(rows_v, [oj])
            gy = plsc.load_gather(rows_v, [oj + WIN])
            dx = p0 - gx
            dy = p1 - gy
            sq = dx * dx + dy * dy
            better = valid & (sq < best_sq)
            best_sq = jnp.where(better, sq, best_sq)
            best_o = jnp.where(better, oj, best_o)

        has = cnt16 > 0
        zero = jnp.zeros((L,), jnp.float32)
        lp = jnp.where(has, 1.0 - jnp.exp(best_sq * NEG_INV_2SIG2), zero)

        g2 = plsc.load_gather(rows_v, [best_o + 2 * WIN])
        g3 = plsc.load_gather(rows_v, [best_o + 3 * WIN])
        g4 = plsc.load_gather(rows_v, [best_o + 4 * WIN])
        g5 = plsc.load_gather(rows_v, [best_o + 5 * WIN])

        la = jnp.where(has, jnp.abs(p2 - g2) + jnp.abs(p3 - g3), zero)
        d = p4 - g4
        ad = jnp.abs(d)
        lw = jnp.where(has, jnp.where(ad < 1.0, 0.5 * d * d, ad - 0.5), zero)
        x = p5
        bce = jnp.maximum(x, zero) - x * g5 + _log1p_series(jnp.exp(-jnp.abs(x)))
        ls = jnp.where(has & (g5 > 0), bce, zero)

        return (s_pos + lp, s_ang + la, s_wid + lw, s_sco + ls, carry_off)

    zeros = jnp.zeros((L,), jnp.float32)
    s_pos, s_ang, s_wid, s_sco, _ = lax.fori_loop(
        0, PW // L, step, (zeros, zeros, zeros, zeros, ws))

    acc_v[pl.ds(0, L)] = s_pos
    acc_v[pl.ds(L, L)] = s_ang
    acc_v[pl.ds(2 * L, L)] = s_wid
    acc_v[pl.ds(3 * L, L)] = s_sco
    pltpu.sync_copy(acc_v, out_hbm.at[pl.ds(wid * 4 * L, 4 * L)])


@jax.jit
def kernel(pred, all_gt_grasps, gt_counts):
    counts = gt_counts.astype(jnp.int32)
    predt = pred.T.reshape(6 * N)
    gtt = all_gt_grasps.T.reshape(6 * M)

    sc_kernel = pl.kernel(
        _body,
        out_type=jax.ShapeDtypeStruct((NW * 4 * L,), jnp.float32),
        mesh=plsc.VectorSubcoreMesh(core_axis_name="c", subcore_axis_name="s",
                                    num_cores=NC, num_subcores=NS),
        compiler_params=pltpu.CompilerParams(needs_layout_passes=False),
        scratch_types=[
            pltpu.VMEM((N,), jnp.int32),
            pltpu.VMEM((6 * PW,), jnp.float32),
            pltpu.VMEM((6 * WIN,), jnp.float32),
            pltpu.VMEM((3 * L // 2,), jnp.int32),
            pltpu.VMEM((4 * L,), jnp.float32),
            pltpu.SemaphoreType.DMA,
        ],
    )
    partials = sc_kernel(gtt, predt, counts)
    sums = partials.reshape(NW, 4, L).sum(axis=(0, 2))
    lp = sums[0] / N
    la = sums[1] / N
    lw = sums[2] / N
    ls = sums[3] / N
    lg = lp + la + lw + 0.5 * ls
    return jnp.stack([lp, la, lw, ls, lg])


# speedup vs baseline: 220.5759x; 1.0398x over previous
import jax
import jax.numpy as jnp
from jax import lax
from jax.experimental import pallas as pl
from jax.experimental.pallas import tpu as pltpu
from jax.experimental.pallas import tpu_sc as plsc

N = 8192
M = 57344
KMAX = 7
NC, NS, L = 2, 16, 16
NW = NC * NS
PW = N // NW
ROWS = PW * KMAX
WIN = ROWS + 8

SIGMA = 0.15
NEG_INV_2SIG2 = -1.0 / (2.0 * SIGMA * SIGMA)


def _log1p_series(z):
    u = z / (2.0 + z)
    u2 = u * u
    p = 1.0 / 9.0
    p = p * u2 + 1.0 / 7.0
    p = p * u2 + 1.0 / 5.0
    p = p * u2 + 1.0 / 3.0
    p = p * u2 + 1.0
    return 2.0 * u * p


def _body(gt_hbm, predt_hbm, off_hbm, cnt_hbm, out_hbm,
          off_v, cnt_v, predt_v, rows_v, acc_v, sem):
    wid = lax.axis_index("s") * NC + lax.axis_index("c")
    base = wid * PW

    d_off = pltpu.async_copy(off_hbm.at[pl.ds(base, PW)], off_v, sem)
    descs = [pltpu.async_copy(cnt_hbm.at[pl.ds(base, PW)], cnt_v, sem)]
    for k in range(6):
        descs.append(pltpu.async_copy(predt_hbm.at[pl.ds(k * N + base, PW)],
                                      predt_v.at[pl.ds(k * PW, PW)], sem))
    d_off.wait()

    ws = off_v[pl.ds(0, L)][0]
    a8 = jnp.minimum((ws // 8) * 8, M - WIN)
    a8 = pl.multiple_of(a8, 8)
    for k in range(6):
        descs.append(pltpu.async_copy(gt_hbm.at[pl.ds(k * M + a8, WIN)],
                                      rows_v.at[pl.ds(k * WIN, WIN)], sem))
    for d in descs:
        d.wait()

    def step(t, carry):
        s_pos, s_ang, s_wid, s_sco = carry
        cnt16 = cnt_v[pl.ds(t * L, L)]
        o = off_v[pl.ds(t * L, L)] - a8
        p0 = predt_v[pl.ds(0 * PW + t * L, L)]
        p1 = predt_v[pl.ds(1 * PW + t * L, L)]
        p2 = predt_v[pl.ds(2 * PW + t * L, L)]
        p3 = predt_v[pl.ds(3 * PW + t * L, L)]
        p4 = predt_v[pl.ds(4 * PW + t * L, L)]
        p5 = predt_v[pl.ds(5 * PW + t * L, L)]

        best_sq = jnp.full((L,), jnp.inf, jnp.float32)
        best_o = jnp.zeros((L,), jnp.int32)
        for j in range(KMAX):
            valid = j < cnt16
            oj = o + j
            gx = plsc.load_gather(rows_v, [oj])
            gy = plsc.load_gather(rows_v, [oj + WIN])
            dx = p0 - gx
            dy = p1 - gy
            sq = dx * dx + dy * dy
            better = valid & (sq < best_sq)
            best_sq = jnp.where(better, sq, best_sq)
            best_o = jnp.where(better, oj, best_o)

        has = cnt16 > 0
        zero = jnp.zeros((L,), jnp.float32)
        lp = jnp.where(has, 1.0 - jnp.exp(best_sq * NEG_INV_2SIG2), zero)

        g2 = plsc.load_gather(rows_v, [best_o + 2 * WIN])
        g3 = plsc.load_gather(rows_v, [best_o + 3 * WIN])
        g4 = plsc.load_gather(rows_v, [best_o + 4 * WIN])
        g5 = plsc.load_gather(rows_v, [best_o + 5 * WIN])

        la = jnp.where(has, jnp.abs(p2 - g2) + jnp.abs(p3 - g3), zero)
        d = p4 - g4
        ad = jnp.abs(d)
        lw = jnp.where(has, jnp.where(ad < 1.0, 0.5 * d * d, ad - 0.5), zero)
        x = p5
        bce = jnp.maximum(x, zero) - x * g5 + _log1p_series(jnp.exp(-jnp.abs(x)))
        ls = jnp.where(has & (g5 > 0), bce, zero)

        return (s_pos + lp, s_ang + la, s_wid + lw, s_sco + ls)

    zeros = jnp.zeros((L,), jnp.float32)
    s_pos, s_ang, s_wid, s_sco = lax.fori_loop(
        0, PW // L, step, (zeros, zeros, zeros, zeros))

    acc_v[pl.ds(0, L)] = s_pos
    acc_v[pl.ds(L, L)] = s_ang
    acc_v[pl.ds(2 * L, L)] = s_wid
    acc_v[pl.ds(3 * L, L)] = s_sco
    wdescs = [
        pltpu.async_copy(acc_v.at[pl.ds(c * L, L)],
                         out_hbm.at[pl.ds(c * NW * L + wid * L, L)], sem)
        for c in range(4)
    ]
    for d in wdescs:
        d.wait()


@jax.jit
def kernel(pred, all_gt_grasps, gt_counts):
    counts = gt_counts.astype(jnp.int32)
    offsets = jnp.cumsum(counts) - counts
    predt = pred.T.reshape(6 * N)
    gtt = all_gt_grasps.T.reshape(6 * M)

    sc_kernel = pl.kernel(
        _body,
        out_type=jax.ShapeDtypeStruct((4 * NW * L,), jnp.float32),
        mesh=plsc.VectorSubcoreMesh(core_axis_name="c", subcore_axis_name="s",
                                    num_cores=NC, num_subcores=NS),
        compiler_params=pltpu.CompilerParams(needs_layout_passes=False),
        scratch_types=[
            pltpu.VMEM((PW,), jnp.int32),
            pltpu.VMEM((PW,), jnp.int32),
            pltpu.VMEM((6 * PW,), jnp.float32),
            pltpu.VMEM((6 * WIN,), jnp.float32),
            pltpu.VMEM((4 * L,), jnp.float32),
            pltpu.SemaphoreType.DMA,
        ],
    )
    partials = sc_kernel(gtt, predt, offsets, counts)
    sums = partials.reshape(4, NW * L).sum(axis=1)
    lp = sums[0] / N
    la = sums[1] / N
    lw = sums[2] / N
    ls = sums[3] / N
    lg = lp + la + lw + 0.5 * ls
    return jnp.stack([lp, la, lw, ls, lg])
